# baseline (speedup 1.0000x reference)
import math
import functools

import jax
import jax.numpy as jnp
from jax import lax
from jax.experimental import pallas as pl
from jax.experimental.pallas import tpu as pltpu
from jax.experimental.pallas import tpu_sc as plsc

D_EMBED = 128
SCALE = math.sqrt(float(D_EMBED))

_NC = 2
_NS = 16
_NW = _NC * _NS
_LANES = 16
_BPC = 2
_NBUF = 8
_LEAD = 6


def _make_gather(n_batch: int, seq_len: int):
    rows_per_chunk = _BPC * seq_len
    assert n_batch % (_NW * _BPC) == 0
    batch_per_w = n_batch // _NW
    n_chunks = batch_per_w // _BPC
    assert n_chunks % _NBUF == 0
    mesh = plsc.VectorSubcoreMesh(core_axis_name="c", subcore_axis_name="s")

    @functools.partial(
        pl.kernel,
        mesh=mesh,
        out_type=jax.ShapeDtypeStruct((n_batch, seq_len, D_EMBED), jnp.float32),
        scratch_types=(
            [
                pltpu.VMEM((n_chunks, rows_per_chunk), jnp.int32),
                pltpu.VMEM((_NBUF, rows_per_chunk, D_EMBED), jnp.float32),
            ]
            + [pltpu.SemaphoreType.DMA] * (2 * _NBUF)
        ),
    )
    def grab(table_hbm, idx_hbm, out_hbm, idx_v, rows_v, *sems):
        gsem = sems[:_NBUF]
        osem = sems[_NBUF:]
        wid = lax.axis_index("s") * _NC + lax.axis_index("c")
        bbase = wid * batch_per_w
        pltpu.sync_copy(idx_hbm.at[wid], idx_v)

        def start_gather(chunk, slot):
            pltpu.async_copy(
                table_hbm.at[idx_v.at[chunk]], rows_v.at[slot], gsem[slot]
            )

        def drain_out(slot):
            for h in range(_BPC):
                pltpu.make_async_copy(
                    rows_v.at[slot, pl.ds(h * seq_len, seq_len)],
                    out_hbm.at[bbase],
                    osem[slot],
                ).wait()

        for b in range(_LEAD):
            start_gather(b, b)

        @pl.loop(0, n_chunks, step=_NBUF)
        def _outer(j0):
            for b in range(_NBUF):
                i = j0 + b
                slot_g = (b + _LEAD) % _NBUF

                @pl.when(i + _LEAD < n_chunks)
                def _():
                    @pl.when(i >= _NBUF - _LEAD)
                    def _():
                        drain_out(slot_g)

                    start_gather(i + _LEAD, slot_g)

                pltpu.make_async_copy(
                    table_hbm.at[idx_v.at[i]], rows_v.at[b], gsem[b]
                ).wait()

                @pl.loop(0, rows_per_chunk, unroll=4)
                def _row(r):
                    for c in range(D_EMBED // _LANES):
                        sl = pl.ds(c * _LANES, _LANES)
                        rows_v[b, r, sl] = rows_v[b, r, sl] * SCALE

                for h in range(_BPC):
                    pltpu.async_copy(
                        rows_v.at[b, pl.ds(h * seq_len, seq_len)],
                        out_hbm.at[bbase + i * _BPC + h],
                        osem[b],
                    )

        for b in range(_NBUF):
            drain_out(b)

    return grab


_NSPLIT = 4


def kernel(x, table):
    n_batch, seq_len = x.shape
    strip = n_batch // _NSPLIT
    grab = _make_gather(strip, seq_len)
    out = jnp.zeros((n_batch, seq_len, D_EMBED), jnp.float32)
    prev = None
    for s in range(_NSPLIT):
        xs = x[s * strip:(s + 1) * strip]
        idx = (
            xs.reshape(-1)
            .astype(jnp.int32)
            .reshape(_NW, -1, _BPC * seq_len)
        )
        if prev is not None:
            idx = idx + (prev[0, 0, 0] * 0.0).astype(jnp.int32)
        o = grab(table, idx)
        prev = o
        out = jax.lax.dynamic_update_slice(out, o, (s * strip, 0, 0))
    return out


# speedup vs baseline: 1.8599x; 1.8599x over previous
import math
import functools

import jax
import jax.numpy as jnp
from jax import lax
from jax.experimental import pallas as pl
from jax.experimental.pallas import tpu as pltpu
from jax.experimental.pallas import tpu_sc as plsc

D_EMBED = 128
SCALE = math.sqrt(float(D_EMBED))

_NC = 2
_NS = 16
_NW = _NC * _NS
_LANES = 16
_BPC = 2
_NBUF = 8
_LEAD = 5


def _make_gather(n_batch: int, seq_len: int):
    rows_per_chunk = _BPC * seq_len
    assert n_batch % (_NW * _BPC) == 0
    batch_per_w = n_batch // _NW
    n_chunks = batch_per_w // _BPC
    assert n_chunks % _NBUF == 0
    mesh = plsc.VectorSubcoreMesh(core_axis_name="c", subcore_axis_name="s")

    @functools.partial(
        pl.kernel,
        mesh=mesh,
        out_type=jax.ShapeDtypeStruct((n_batch, seq_len, D_EMBED), jnp.float32),
        scratch_types=(
            [
                pltpu.VMEM((n_chunks, rows_per_chunk), jnp.int32),
                pltpu.VMEM((_NBUF, rows_per_chunk, D_EMBED), jnp.float32),
            ]
            + [pltpu.SemaphoreType.DMA] * (2 * _NBUF)
        ),
    )
    def grab(table_hbm, idx_hbm, out_hbm, idx_v, rows_v, *sems):
        gsem = sems[:_NBUF]
        osem = sems[_NBUF:]
        wid = lax.axis_index("s") * _NC + lax.axis_index("c")
        bbase = wid * batch_per_w
        pltpu.sync_copy(idx_hbm.at[wid], idx_v)

        def start_gather(chunk, slot):
            pltpu.async_copy(
                table_hbm.at[idx_v.at[chunk]], rows_v.at[slot], gsem[slot]
            )

        def drain_out(slot):
            for h in range(_BPC):
                pltpu.make_async_copy(
                    rows_v.at[slot, pl.ds(h * seq_len, seq_len)],
                    out_hbm.at[bbase],
                    osem[slot],
                ).wait()

        for b in range(_LEAD):
            start_gather(b, b)

        @pl.loop(0, n_chunks, step=_NBUF)
        def _outer(j0):
            for b in range(_NBUF):
                i = j0 + b
                slot_g = (b + _LEAD) % _NBUF

                @pl.when(i + _LEAD < n_chunks)
                def _():
                    @pl.when(i >= _NBUF - _LEAD)
                    def _():
                        drain_out(slot_g)

                    start_gather(i + _LEAD, slot_g)

                pltpu.make_async_copy(
                    table_hbm.at[idx_v.at[i]], rows_v.at[b], gsem[b]
                ).wait()

                @pl.loop(0, rows_per_chunk)
                def _row(r):
                    for c in range(D_EMBED // _LANES):
                        sl = pl.ds(c * _LANES, _LANES)
                        rows_v[b, r, sl] = rows_v[b, r, sl] * SCALE

                for h in range(_BPC):
                    pltpu.async_copy(
                        rows_v.at[b, pl.ds(h * seq_len, seq_len)],
                        out_hbm.at[bbase + i * _BPC + h],
                        osem[b],
                    )

        for b in range(_NBUF):
            drain_out(b)

    return grab


def kernel(x, table):
    n_batch, seq_len = x.shape
    idx = (
        x.reshape(-1)
        .astype(jnp.int32)
        .reshape(_NW, -1, _BPC * seq_len)
    )
    return _make_gather(n_batch, seq_len)(table, idx)
